# baseline (speedup 1.0000x reference)
import functools

import jax
import jax.numpy as jnp
from jax import lax
from jax.experimental import pallas as pl
from jax.experimental.pallas import tpu as pltpu
from jax.experimental.pallas import tpu_sc as plsc

B = 16384
F = 14
V = 100000
D = 32
NUM_NUMERIC = 64
NUM_CLASSES = 2

NC = 2
NS = 16
NW = NC * NS
ROWS_PW = B // NW
C = 128
NCHUNK = ROWS_PW // C

VP = 100096
PLANE = D * VP

GROUPS = ((0, 2), (2, 5), (7, 7))

_mesh = plsc.VectorSubcoreMesh(core_axis_name="c", subcore_axis_name="s")


def _strip_body(in_ref, out_ref):
    for r in range(8):
        out_ref[pl.ds(r * VP, V)] = in_ref[0, r, :]


def _strip(tab_t, f0, nf):
    return pl.pallas_call(
        _strip_body,
        grid=(nf, D // 8),
        in_specs=[pl.BlockSpec((1, 8, V), lambda f, j: (f0 + f, j, 0))],
        out_specs=pl.BlockSpec((8 * VP,), lambda f, j: (f * (D // 8) + j,)),
        out_shape=jax.ShapeDtypeStruct((nf * PLANE,), jnp.float32),
    )(tab_t)


def _make_sc_embed(f0, nf):
    @functools.partial(
        pl.kernel,
        mesh=_mesh,
        compiler_params=pltpu.CompilerParams(use_tc_tiling_on_sc=False),
        out_type=jax.ShapeDtypeStruct((D, B), jnp.float32),
        scratch_types=[
            pltpu.VMEM((F, ROWS_PW), jnp.int32),
            pltpu.VMEM((2, D * C), jnp.float32),
            pltpu.VMEM((D, C), jnp.float32),
            pltpu.SemaphoreType.DMA((2,)),
        ],
    )
    def _sc_embed(xcat_hbm, tab_hbm, out_hbm, xcat_v, gat_v, emb_v, sem):
        wid = lax.axis_index("s") * NC + lax.axis_index("c")
        b0 = wid * ROWS_PW
        pltpu.sync_copy(xcat_hbm.at[:, pl.ds(b0, ROWS_PW)], xcat_v)
        ng = NCHUNK * nf

        def _fire(g, par):
            ch = g // nf
            f = g - ch * nf
            idx = xcat_v.at[f0 + f, pl.ds(ch * C, C)]
            base = f * PLANE
            for d in range(D):
                pltpu.async_copy(
                    tab_hbm.at[pl.ds(base + d * VP, V)].at[idx],
                    gat_v.at[par, pl.ds(d * C, C)], sem.at[par])

        _fire(0, 0)

        def _group(g, carry):
            ch = g // nf
            f = g - ch * nf
            par = lax.rem(g, 2)

            @pl.when(g < ng - 1)
            def _():
                _fire(g + 1, 1 - par)

            @pl.when(f == 0)
            def _():
                for d in range(D):
                    for k in range(C // 16):
                        emb_v[d, pl.ds(k * 16, 16)] = jnp.zeros((16,),
                                                                jnp.float32)

            pltpu.make_async_copy(tab_hbm.at[pl.ds(0, D * C)],
                                  gat_v.at[par], sem.at[par]).wait()
            for d in range(D):
                for k in range(C // 16):
                    s = pl.ds(k * 16, 16)
                    plsc.addupdate(emb_v.at[d, s],
                                   gat_v[par, pl.ds(d * C + k * 16, 16)])

            @pl.when(f == nf - 1)
            def _():
                pltpu.sync_copy(emb_v, out_hbm.at[:, pl.ds(b0 + ch * C, C)])
            return carry

        lax.fori_loop(0, ng, _group, 0)

    return _sc_embed


_sc_embeds = tuple(_make_sc_embed(f0, nf) for f0, nf in GROUPS)

_BLK = 2048


def _dense_body(ea_ref, eb_ref, ec_ref, xn_ref, wn_ref, bn_ref, wo_ref,
                bo_ref, out_ref):
    h = jnp.dot(wn_ref[...], xn_ref[...], preferred_element_type=jnp.float32)
    h = h + bn_ref[...] + ea_ref[...] + eb_ref[...] + ec_ref[...]
    out_ref[...] = jnp.dot(wo_ref[...], h,
                           preferred_element_type=jnp.float32) + bo_ref[...]


def _dense(embs, xn_t, WnT, b_num, WoT, b_out):
    grid = (B // _BLK,)
    return pl.pallas_call(
        _dense_body,
        grid=grid,
        in_specs=[
            pl.BlockSpec((D, _BLK), lambda i: (0, i)),
            pl.BlockSpec((D, _BLK), lambda i: (0, i)),
            pl.BlockSpec((D, _BLK), lambda i: (0, i)),
            pl.BlockSpec((NUM_NUMERIC, _BLK), lambda i: (0, i)),
            pl.BlockSpec((D, NUM_NUMERIC), lambda i: (0, 0)),
            pl.BlockSpec((D, 1), lambda i: (0, 0)),
            pl.BlockSpec((NUM_CLASSES, D), lambda i: (0, 0)),
            pl.BlockSpec((NUM_CLASSES, 1), lambda i: (0, 0)),
        ],
        out_specs=pl.BlockSpec((NUM_CLASSES, _BLK), lambda i: (0, i)),
        out_shape=jax.ShapeDtypeStruct((NUM_CLASSES, B), jnp.float32),
    )(*embs, xn_t, WnT, b_num, WoT, b_out)


def kernel(x_cat, x_num, tables, W_num, b_num, W_out, b_out):
    xcat_t = x_cat.T
    tab_t = tables.transpose(0, 2, 1)
    embs = []
    src = tab_t
    for gi, (f0, nf) in enumerate(GROUPS):
        tab_g = _strip(src, f0, nf)
        embs.append(_sc_embeds[gi](xcat_t, tab_g))
        src = lax.optimization_barrier((tab_t, tab_g))[0]
    out_t = _dense(embs, x_num.T, W_num.T, b_num.reshape(D, 1),
                   W_out.T, b_out.reshape(NUM_CLASSES, 1))
    return out_t.T


# speedup vs baseline: 1.0028x; 1.0028x over previous
import functools

import jax
import jax.numpy as jnp
from jax import lax
from jax.experimental import pallas as pl
from jax.experimental.pallas import tpu as pltpu
from jax.experimental.pallas import tpu_sc as plsc

B = 16384
F = 14
V = 100000
D = 32
NUM_NUMERIC = 64
NUM_CLASSES = 2

NC = 2
NS = 16
NW = NC * NS
ROWS_PW = B // NW
C = 128
NCHUNK = ROWS_PW // C

VP = 100096
PLANE = D * VP

GROUPS = ((0, 1), (1, 4), (5, 9))

_mesh = plsc.VectorSubcoreMesh(core_axis_name="c", subcore_axis_name="s")


def _strip_body(in_ref, out_ref):
    for r in range(8):
        out_ref[pl.ds(r * VP, V)] = in_ref[0, r, :]


def _strip(tab_t, f0, nf):
    return pl.pallas_call(
        _strip_body,
        grid=(nf, D // 8),
        in_specs=[pl.BlockSpec((1, 8, V), lambda f, j: (f0 + f, j, 0))],
        out_specs=pl.BlockSpec((8 * VP,), lambda f, j: (f * (D // 8) + j,)),
        out_shape=jax.ShapeDtypeStruct((nf * PLANE,), jnp.float32),
    )(tab_t)


def _make_sc_embed(f0, nf):
    @functools.partial(
        pl.kernel,
        mesh=_mesh,
        compiler_params=pltpu.CompilerParams(use_tc_tiling_on_sc=False),
        out_type=jax.ShapeDtypeStruct((D, B), jnp.float32),
        scratch_types=[
            pltpu.VMEM((F, ROWS_PW), jnp.int32),
            pltpu.VMEM((2, D * C), jnp.float32),
            pltpu.VMEM((D, C), jnp.float32),
            pltpu.SemaphoreType.DMA((2,)),
        ],
    )
    def _sc_embed(xcat_hbm, tab_hbm, out_hbm, xcat_v, gat_v, emb_v, sem):
        wid = lax.axis_index("s") * NC + lax.axis_index("c")
        b0 = wid * ROWS_PW
        pltpu.sync_copy(xcat_hbm.at[:, pl.ds(b0, ROWS_PW)], xcat_v)
        ng = NCHUNK * nf

        def _fire(g, par):
            ch = g // nf
            f = g - ch * nf
            idx = xcat_v.at[f0 + f, pl.ds(ch * C, C)]
            base = f * PLANE
            for d in range(D):
                pltpu.async_copy(
                    tab_hbm.at[pl.ds(base + d * VP, V)].at[idx],
                    gat_v.at[par, pl.ds(d * C, C)], sem.at[par])

        _fire(0, 0)

        def _group(g, carry):
            ch = g // nf
            f = g - ch * nf
            par = lax.rem(g, 2)

            @pl.when(g < ng - 1)
            def _():
                _fire(g + 1, 1 - par)

            @pl.when(f == 0)
            def _():
                for d in range(D):
                    for k in range(C // 16):
                        emb_v[d, pl.ds(k * 16, 16)] = jnp.zeros((16,),
                                                                jnp.float32)

            pltpu.make_async_copy(tab_hbm.at[pl.ds(0, D * C)],
                                  gat_v.at[par], sem.at[par]).wait()
            for d in range(D):
                for k in range(C // 16):
                    s = pl.ds(k * 16, 16)
                    plsc.addupdate(emb_v.at[d, s],
                                   gat_v[par, pl.ds(d * C + k * 16, 16)])

            @pl.when(f == nf - 1)
            def _():
                pltpu.sync_copy(emb_v, out_hbm.at[:, pl.ds(b0 + ch * C, C)])
            return carry

        lax.fori_loop(0, ng, _group, 0)

    return _sc_embed


_sc_embeds = tuple(_make_sc_embed(f0, nf) for f0, nf in GROUPS)

_BLK = 2048


def _dense_body(ea_ref, eb_ref, ec_ref, xn_ref, wn_ref, bn_ref, wo_ref,
                bo_ref, out_ref):
    h = jnp.dot(wn_ref[...], xn_ref[...], preferred_element_type=jnp.float32)
    h = h + bn_ref[...] + ea_ref[...] + eb_ref[...] + ec_ref[...]
    out_ref[...] = jnp.dot(wo_ref[...], h,
                           preferred_element_type=jnp.float32) + bo_ref[...]


def _dense(embs, xn_t, WnT, b_num, WoT, b_out):
    grid = (B // _BLK,)
    return pl.pallas_call(
        _dense_body,
        grid=grid,
        in_specs=[
            pl.BlockSpec((D, _BLK), lambda i: (0, i)),
            pl.BlockSpec((D, _BLK), lambda i: (0, i)),
            pl.BlockSpec((D, _BLK), lambda i: (0, i)),
            pl.BlockSpec((NUM_NUMERIC, _BLK), lambda i: (0, i)),
            pl.BlockSpec((D, NUM_NUMERIC), lambda i: (0, 0)),
            pl.BlockSpec((D, 1), lambda i: (0, 0)),
            pl.BlockSpec((NUM_CLASSES, D), lambda i: (0, 0)),
            pl.BlockSpec((NUM_CLASSES, 1), lambda i: (0, 0)),
        ],
        out_specs=pl.BlockSpec((NUM_CLASSES, _BLK), lambda i: (0, i)),
        out_shape=jax.ShapeDtypeStruct((NUM_CLASSES, B), jnp.float32),
    )(*embs, xn_t, WnT, b_num, WoT, b_out)


def kernel(x_cat, x_num, tables, W_num, b_num, W_out, b_out):
    xcat_t = x_cat.T
    tab_t = tables.transpose(0, 2, 1)
    embs = []
    src = tab_t
    for gi, (f0, nf) in enumerate(GROUPS):
        tab_g = _strip(src, f0, nf)
        embs.append(_sc_embeds[gi](xcat_t, tab_g))
        src = lax.optimization_barrier((tab_t, tab_g))[0]
    out_t = _dense(embs, x_num.T, W_num.T, b_num.reshape(D, 1),
                   W_out.T, b_out.reshape(NUM_CLASSES, 1))
    return out_t.T
